# baseline (speedup 1.0000x reference)
import functools

import jax
import jax.numpy as jnp
from jax import lax
from jax.experimental import pallas as pl
from jax.experimental.pallas import tpu as pltpu
from jax.experimental.pallas import tpu_sc as plsc

NUM_HEADS = 64
K_BITS = 16
V_BITS = 16
NEURONS = 8
NBN_STATE = 12
NBN_OUT = 8
WIN = K_BITS + V_BITS
NWIN = 4096
NSUB = 16
WPT = NWIN // NSUB
GPT = WPT // 16


def _body(bits_hbm, smem_hbm, omem_hbm, cst_hbm, cot_hbm, out_hbm,
          win_v, krev_l, head_l, wr_l, stage_l, seg_l, mq_v, hk_v,
          seg_v, cnt_v, sm_v, om_v, cst_v, cot_v, out_v,
          mq_s, hk_s, seg_s, cnt_s, sem):
    cid = lax.axis_index("c")
    sid = lax.axis_index("s")

    @pl.when(cid == 0)
    def _route():
        iota = lax.iota(jnp.int32, 16)
        base = iota * WIN
        w0_sub = sid * WPT

        with jax.named_scope("p1_dma"):
            pltpu.sync_copy(bits_hbm.at[pl.ds(w0_sub * WIN, WPT * WIN)], win_v)

        def group_body(g, maxq):
            boff = base + g * (16 * WIN)
            krev = plsc.load_gather(win_v, [boff])
            headv = jnp.zeros((16,), jnp.int32)
            for i in range(1, K_BITS):
                b = plsc.load_gather(win_v, [boff + i])
                krev = krev | (b << i)
                if i >= 10:
                    headv = headv | (b << (15 - i))
            wrv = plsc.load_gather(win_v, [boff + K_BITS])
            for i in range(K_BITS + 1, WIN):
                wrv = wrv | plsc.load_gather(win_v, [boff + i])
            off = g * 16
            krev_l[pl.ds(off, 16)] = krev
            head_l[pl.ds(off, 16)] = headv
            wr_l[pl.ds(off, 16)] = wrv
            widx = iota + (w0_sub + off)
            return jnp.maximum(maxq, jnp.where(wrv > 0, -1, widx))

        with jax.named_scope("p1_route"):
            maxq = lax.fori_loop(0, GPT, group_body, jnp.full((16,), -1, jnp.int32))
        with jax.named_scope("p1_pub"):
            stage_l[...] = maxq
            pltpu.sync_copy(stage_l, mq_s.at[pl.ds(sid * 16, 16)])
            plsc.subcore_barrier()

        pltpu.sync_copy(mq_s, mq_v)

        def mq_body(i, mq):
            return jnp.maximum(mq, mq_v[pl.ds(i * 16, 16)])
        mq = lax.fori_loop(0, NSUB, mq_body, jnp.full((16,), -1, jnp.int32))
        wq = jnp.max(mq)
        wstar = jnp.where(wq < 0, NWIN - 1, wq).astype(jnp.int32)

        @pl.when(wstar // WPT == sid)
        def _publish_hk():
            lidx = jnp.full((16,), wstar - w0_sub, jnp.int32)
            stage_l[...] = plsc.load_gather(head_l, [lidx])
            pltpu.sync_copy(stage_l, hk_s.at[pl.ds(0, 16)])
            stage_l[...] = plsc.load_gather(krev_l, [lidx])
            pltpu.sync_copy(stage_l, hk_s.at[pl.ds(16, 16)])
        plsc.subcore_barrier()

        pltpu.sync_copy(hk_s, hk_v)
        hstar = hk_v[pl.ds(0, 16)][0]

        @pl.when(sid == 0)
        def _fire_dma():
            pltpu.async_copy(smem_hbm.at[hstar], sm_v, sem)
            pltpu.async_copy(omem_hbm.at[hstar], om_v, sem)
            pltpu.async_copy(cst_hbm.at[hstar], cst_v, sem)
            pltpu.async_copy(cot_hbm.at[hstar], cot_v, sem)

        def comp_body(g, cnt):
            off = g * 16
            kr = krev_l[pl.ds(off, 16)]
            hd = head_l[pl.ds(off, 16)]
            wr = wr_l[pl.ds(off, 16)]
            widx = iota + (w0_sub + off)
            m = jnp.logical_and(jnp.logical_and(hd == hstar, wr > 0),
                                widx < wstar)
            plsc.store_compressed(seg_l.at[pl.ds(cnt, 16)], kr, mask=m)
            return cnt + jnp.sum(m.astype(jnp.int32))

        with jax.named_scope("p3_compact"):
            cnt = lax.fori_loop(0, GPT, comp_body, jnp.int32(0))
        stage_l[...] = jnp.full((16,), cnt, jnp.int32)
        pltpu.sync_copy(stage_l, cnt_s.at[pl.ds(sid * 16, 16)])
        pltpu.sync_copy(seg_l.at[pl.ds(0, WPT)], seg_s.at[pl.ds(sid * WPT, WPT)])
        plsc.subcore_barrier()

    @pl.when(jnp.logical_and(cid == 0, sid == 0))
    def _tail():
        iota = lax.iota(jnp.int32, 16)
        with jax.named_scope("p4_stage"):
            pltpu.sync_copy(seg_s, seg_v.at[pl.ds(0, NWIN)])
            pltpu.sync_copy(cnt_s, cnt_v)
        krev_star = hk_v[pl.ds(16, 16)][0]

        pltpu.make_async_copy(smem_hbm.at[0], sm_v, sem).wait()
        pltpu.make_async_copy(omem_hbm.at[0], om_v, sem).wait()
        pltpu.make_async_copy(cst_hbm.at[0], cst_v, sem).wait()
        pltpu.make_async_copy(cot_hbm.at[0], cot_v, sem).wait()

        nid = iota & 7
        cs = [plsc.load_gather(cst_v, [nid, jnp.full((16,), j, jnp.int32)])
              for j in range(NBN_STATE)]
        lane_lt8 = iota < 8

        def new_state_mask(inp_mask):
            inp_v = jnp.full((16,), inp_mask, jnp.int32)
            addr = (lax.shift_right_logical(inp_v, cs[0]) & 1)
            for j in range(1, NBN_STATE):
                addr = addr | ((lax.shift_right_logical(inp_v, cs[j]) & 1) << j)
            vals = plsc.load_gather(sm_v, [nid, addr])
            bits = jnp.logical_and(vals > 0.5, lane_lt8).astype(jnp.int32)
            return jnp.sum(bits << iota)

        def seg_body(i, smask0):
            cnt_i = cnt_v[pl.ds(i * 16, 16)][0]
            off = i * WPT

            def chain_body(t, smask):
                krev_t = seg_v[pl.ds(off + t, 16)][0]
                return new_state_mask(krev_t | (smask << K_BITS))
            return lax.fori_loop(0, cnt_i, chain_body, smask0)

        with jax.named_scope("p4_chain"):
            smask = lax.fori_loop(0, NSUB, seg_body, jnp.int32(0))

        s = new_state_mask(krev_star | (smask << K_BITS))
        sv = jnp.full((16,), s, jnp.int32)
        co = [plsc.load_gather(cot_v, [iota, jnp.full((16,), j, jnp.int32)])
              for j in range(NBN_OUT)]
        addr2 = lax.shift_right_logical(sv, co[0]) & 1
        for j in range(1, NBN_OUT):
            addr2 = addr2 | ((lax.shift_right_logical(sv, co[j]) & 1) << j)
        out_v[...] = plsc.load_gather(om_v, [iota, addr2])
        pltpu.sync_copy(out_v, out_hbm)


_sc_call = functools.partial(
    pl.kernel,
    out_type=jax.ShapeDtypeStruct((V_BITS,), jnp.float32),
    mesh=plsc.VectorSubcoreMesh(core_axis_name="c", subcore_axis_name="s"),
    scratch_types=[
        pltpu.VMEM((WPT * WIN,), jnp.int32),
        pltpu.VMEM((WPT,), jnp.int32),
        pltpu.VMEM((WPT,), jnp.int32),
        pltpu.VMEM((WPT,), jnp.int32),
        pltpu.VMEM((16,), jnp.int32),
        pltpu.VMEM((WPT + 16,), jnp.int32),
        pltpu.VMEM((NSUB * 16,), jnp.int32),
        pltpu.VMEM((32,), jnp.int32),
        pltpu.VMEM((NWIN + 16,), jnp.int32),
        pltpu.VMEM((NSUB * 16,), jnp.int32),
        pltpu.VMEM((NEURONS, 2 ** NBN_STATE), jnp.float32),
        pltpu.VMEM((V_BITS, 2 ** NBN_OUT), jnp.float32),
        pltpu.VMEM((NEURONS, NBN_STATE), jnp.int32),
        pltpu.VMEM((V_BITS, NBN_OUT), jnp.int32),
        pltpu.VMEM((V_BITS,), jnp.float32),
        pltpu.VMEM_SHARED((NSUB * 16,), jnp.int32),
        pltpu.VMEM_SHARED((32,), jnp.int32),
        pltpu.VMEM_SHARED((NWIN,), jnp.int32),
        pltpu.VMEM_SHARED((NSUB * 16,), jnp.int32),
        pltpu.SemaphoreType.DMA,
    ],
    compiler_params=pltpu.CompilerParams(needs_layout_passes=False),
)(_body)


def kernel(input_bits, state_memory, output_memory, conn_state, conn_out):
    return _sc_call(input_bits.astype(jnp.int32), state_memory, output_memory,
                    conn_state.astype(jnp.int32), conn_out.astype(jnp.int32))


# speedup vs baseline: 1.0193x; 1.0193x over previous
import functools

import jax
import jax.numpy as jnp
from jax import lax
from jax.experimental import pallas as pl
from jax.experimental.pallas import tpu as pltpu
from jax.experimental.pallas import tpu_sc as plsc

NUM_HEADS = 64
K_BITS = 16
V_BITS = 16
NEURONS = 8
NBN_STATE = 12
NBN_OUT = 8
WIN = K_BITS + V_BITS
NWIN = 4096
NSUB = 16
WPT = NWIN // NSUB
GPT = WPT // 16


def _or_tree(terms):
    while len(terms) > 1:
        terms = [a | b for a, b in zip(terms[::2], terms[1::2])] + (
            [terms[-1]] if len(terms) % 2 else [])
    return terms[0]


def _body(bits_hbm, smem_hbm, omem_hbm, cst_hbm, cot_hbm, out_hbm,
          win_v, krev_l, head_l, wr_l, stage_l, seg_l, rec_v,
          seg_v, cnt_v, sm_v, om_v, cst_v, cot_v, out_v,
          rec_s, seg_s, cnt_s, sem):
    cid = lax.axis_index("c")
    sid = lax.axis_index("s")

    @pl.when(cid == 0)
    def _run():
        iota = lax.iota(jnp.int32, 16)
        base = iota * WIN
        w0_sub = sid * WPT

        pltpu.sync_copy(bits_hbm.at[pl.ds(w0_sub * WIN, WPT * WIN)], win_v)

        maxq = jnp.full((16,), -1, jnp.int32)
        for g in range(GPT):
            boff = base + g * (16 * WIN)
            kb = [plsc.load_gather(win_v, [boff + i]) for i in range(K_BITS)]
            vb = [plsc.load_gather(win_v, [boff + K_BITS + i])
                  for i in range(V_BITS)]
            krev = _or_tree([kb[i] << i for i in range(K_BITS)])
            headv = _or_tree([kb[i] << (15 - i) for i in range(10, K_BITS)])
            wrv = _or_tree(vb)
            off = g * 16
            krev_l[pl.ds(off, 16)] = krev
            head_l[pl.ds(off, 16)] = headv
            wr_l[pl.ds(off, 16)] = wrv
            widx = iota + (w0_sub + off)
            maxq = jnp.maximum(maxq, jnp.where(wrv > 0, -1, widx))

        wcand = jnp.max(maxq)
        lidxq = jnp.maximum(wcand - w0_sub, 0)
        headq = plsc.load_gather(head_l, [jnp.full((16,), lidxq, jnp.int32)])[0]
        krevq = plsc.load_gather(krev_l, [jnp.full((16,), lidxq, jnp.int32)])[0]
        headlast = head_l[pl.ds(WPT - 16, 16)][15]
        krevlast = krev_l[pl.ds(WPT - 16, 16)][15]
        rec = jnp.where(iota == 0, wcand,
                        jnp.where(iota == 1, headq,
                                  jnp.where(iota == 2, krevq,
                                            jnp.where(iota == 3, headlast,
                                                      krevlast))))
        stage_l[...] = rec
        pltpu.sync_copy(stage_l, rec_s.at[pl.ds(sid * 16, 16)])
        plsc.subcore_barrier()

        pltpu.sync_copy(rec_s, rec_v)

        def red_body(i, carry):
            bw, bh, bk = carry
            rv = rec_v[pl.ds(i * 16, 16)]
            wc = rv[0]
            upd = wc > bw
            return (jnp.where(upd, wc, bw), jnp.where(upd, rv[1], bh),
                    jnp.where(upd, rv[2], bk))

        bw, bh, bk = lax.fori_loop(
            0, NSUB, red_body,
            (jnp.int32(-1), jnp.int32(0), jnp.int32(0)))
        r15 = rec_v[pl.ds((NSUB - 1) * 16, 16)]
        no_q = bw < 0
        wstar = jnp.where(no_q, NWIN - 1, bw)
        hstar = jnp.where(no_q, r15[3], bh)
        krev_star = jnp.where(no_q, r15[4], bk)

        @pl.when(sid == 0)
        def _fire_dma():
            pltpu.async_copy(smem_hbm.at[hstar], sm_v, sem)
            pltpu.async_copy(omem_hbm.at[hstar], om_v, sem)
            pltpu.async_copy(cst_hbm.at[hstar], cst_v, sem)
            pltpu.async_copy(cot_hbm.at[hstar], cot_v, sem)

        def comp_body(g, cnt):
            off = g * 16
            kr = krev_l[pl.ds(off, 16)]
            hd = head_l[pl.ds(off, 16)]
            wr = wr_l[pl.ds(off, 16)]
            widx = iota + (w0_sub + off)
            m = jnp.logical_and(jnp.logical_and(hd == hstar, wr > 0),
                                widx < wstar)
            plsc.store_compressed(seg_l.at[pl.ds(cnt, 16)], kr, mask=m)
            return cnt + jnp.sum(m.astype(jnp.int32))

        cnt = lax.fori_loop(0, GPT, comp_body, jnp.int32(0))
        stage_l[...] = jnp.full((16,), cnt, jnp.int32)
        pltpu.sync_copy(stage_l, cnt_s.at[pl.ds(sid * 16, 16)])
        pltpu.sync_copy(seg_l.at[pl.ds(0, WPT)], seg_s.at[pl.ds(sid * WPT, WPT)])
        plsc.subcore_barrier()

        @pl.when(sid == 0)
        def _tail():
            pltpu.sync_copy(seg_s, seg_v.at[pl.ds(0, NWIN)])
            pltpu.sync_copy(cnt_s, cnt_v)

            pltpu.make_async_copy(smem_hbm.at[0], sm_v, sem).wait()
            pltpu.make_async_copy(omem_hbm.at[0], om_v, sem).wait()
            pltpu.make_async_copy(cst_hbm.at[0], cst_v, sem).wait()
            pltpu.make_async_copy(cot_hbm.at[0], cot_v, sem).wait()

            nid = iota & 7
            zid = iota * 0
            cs = [plsc.load_gather(cst_v, [nid, jnp.full((16,), j, jnp.int32)])
                  for j in range(NBN_STATE)]
            sel = [c < K_BITS for c in cs]
            csk = [jnp.minimum(c, K_BITS - 1) for c in cs]
            css = [jnp.maximum(c - K_BITS, 0) for c in cs]
            co = [plsc.load_gather(cot_v, [iota, jnp.full((16,), j, jnp.int32)])
                  for j in range(NBN_OUT)]

            def new_state_vec(key_vec, state_vec):
                terms = []
                for j in range(NBN_STATE):
                    kbit = key_vec.at[csk[j]].get(mode="promise_in_bounds")
                    sbit = state_vec.at[css[j]].get(mode="promise_in_bounds")
                    terms.append(jnp.where(sel[j], kbit, sbit) << j)
                addr = _or_tree(terms)
                vals = plsc.load_gather(sm_v, [nid, addr])
                return (vals > 0.5).astype(jnp.int32)

            def seg_body(i, state0):
                cnt_i = cnt_v[pl.ds(i * 16, 16)][0]
                off = i * WPT

                def chain_body(t, state):
                    seg16 = seg_v[pl.ds(off + t, 16)]
                    krev_b = seg16.at[zid].get(mode="promise_in_bounds")
                    key_vec = (krev_b >> iota) & 1
                    return new_state_vec(key_vec, state)
                return lax.fori_loop(0, cnt_i, chain_body, state0)

            state = lax.fori_loop(0, NSUB, seg_body,
                                  jnp.zeros((16,), jnp.int32))

            key_star = (jnp.full((16,), krev_star, jnp.int32) >> iota) & 1
            ns = new_state_vec(key_star, state)
            terms2 = [(ns.at[co[j]].get(mode="promise_in_bounds") << j)
                      for j in range(NBN_OUT)]
            addr2 = _or_tree(terms2)
            out_v[...] = plsc.load_gather(om_v, [iota, addr2])
            pltpu.sync_copy(out_v, out_hbm)


_sc_call = functools.partial(
    pl.kernel,
    out_type=jax.ShapeDtypeStruct((V_BITS,), jnp.float32),
    mesh=plsc.VectorSubcoreMesh(core_axis_name="c", subcore_axis_name="s"),
    scratch_types=[
        pltpu.VMEM((WPT * WIN,), jnp.int32),
        pltpu.VMEM((WPT,), jnp.int32),
        pltpu.VMEM((WPT,), jnp.int32),
        pltpu.VMEM((WPT,), jnp.int32),
        pltpu.VMEM((16,), jnp.int32),
        pltpu.VMEM((WPT + 16,), jnp.int32),
        pltpu.VMEM((NSUB * 16,), jnp.int32),
        pltpu.VMEM((NWIN + 16,), jnp.int32),
        pltpu.VMEM((NSUB * 16,), jnp.int32),
        pltpu.VMEM((NEURONS, 2 ** NBN_STATE), jnp.float32),
        pltpu.VMEM((V_BITS, 2 ** NBN_OUT), jnp.float32),
        pltpu.VMEM((NEURONS, NBN_STATE), jnp.int32),
        pltpu.VMEM((V_BITS, NBN_OUT), jnp.int32),
        pltpu.VMEM((V_BITS,), jnp.float32),
        pltpu.VMEM_SHARED((NSUB * 16,), jnp.int32),
        pltpu.VMEM_SHARED((NWIN,), jnp.int32),
        pltpu.VMEM_SHARED((NSUB * 16,), jnp.int32),
        pltpu.SemaphoreType.DMA,
    ],
    compiler_params=pltpu.CompilerParams(needs_layout_passes=False),
)(_body)


def kernel(input_bits, state_memory, output_memory, conn_state, conn_out):
    return _sc_call(input_bits.astype(jnp.int32), state_memory, output_memory,
                    conn_state.astype(jnp.int32), conn_out.astype(jnp.int32))
